# Initial kernel scaffold; baseline (speedup 1.0000x reference)
import jax
import jax.numpy as jnp
from jax.experimental import pallas as pl


def kernel(x, edge_index, edge_type, Wq, bq, Wk, bk, Wv, bv, Ws, bs, W_att, a_sem):
    raise NotImplementedError("write your pallas kernel here")


# speedup vs baseline: 7.4988x; 7.4988x over previous
import dataclasses
import functools

import jax
import jax.numpy as jnp
from jax import lax
from jax.experimental import pallas as pl
from jax.experimental.pallas import tpu as pltpu
from jax.experimental.pallas import tpu_sc as plsc

N = 10000
E = 320000
D = 128
R = 2
SH = 4

NC = 2
NS = 16
NW = NC * NS
LB = 128
EP = 327680
PER_TILE_B = EP // NW
PER_SUB_D = EP // NS
ACC_ROWS = 20480
STRIPE = ACC_ROWS // NS
DUMMY = 2 * N
AW = 80

_BN = 1000
_BC = 2048


def _proj_body(x_ref, w_ref, b_ref, o_ref):
    o_ref[0, 0] = (
        jnp.dot(x_ref[...], w_ref[0, 0], preferred_element_type=jnp.float32)
        + b_ref[0, 0]
    )


def _projections(x, Wall, ball):
    return pl.pallas_call(
        _proj_body,
        grid=(4, R, N // _BN),
        in_specs=[
            pl.BlockSpec((_BN, D), lambda k, r, i: (i, 0)),
            pl.BlockSpec((1, 1, D, D), lambda k, r, i: (k, r, 0, 0)),
            pl.BlockSpec((1, 1, 1, D), lambda k, r, i: (k, r, 0, 0)),
        ],
        out_specs=pl.BlockSpec((1, 1, _BN, D), lambda k, r, i: (k, r, i, 0)),
        out_shape=jax.ShapeDtypeStruct((4, R, N, D), jnp.float32),
    )(x, Wall, ball)


def _gather_qk_body(qf_hbm, kf_hbm, gq_hbm, gs_hbm, qd_hbm, ks_hbm,
                    qi_v, si_v, qrows, krows, sem_q, sem_k):
    cid = lax.axis_index("c")
    sid = lax.axis_index("s")
    base0 = (sid * NC + cid) * PER_TILE_B

    @pl.loop(0, PER_TILE_B // LB)
    def _(b):
        base = base0 + b * LB
        pltpu.sync_copy(gq_hbm.at[pl.ds(base, LB)], qi_v)
        pltpu.sync_copy(gs_hbm.at[pl.ds(base, LB)], si_v)
        cq = pltpu.async_copy(qf_hbm.at[qi_v], qrows, sem_q)
        ck = pltpu.async_copy(kf_hbm.at[si_v], krows, sem_k)
        cq.wait()
        ck.wait()
        pltpu.sync_copy(qrows, qd_hbm.at[pl.ds(base, LB)])
        pltpu.sync_copy(krows, ks_hbm.at[pl.ds(base, LB)])


def _gather_qk(qf, kf, gidx_g, sidx_g):
    mesh = plsc.VectorSubcoreMesh(core_axis_name="c", subcore_axis_name="s")
    kern = pl.kernel(
        _gather_qk_body,
        mesh=mesh,
        out_type=[
            jax.ShapeDtypeStruct((EP, D), jnp.float32),
            jax.ShapeDtypeStruct((EP, D), jnp.float32),
        ],
        scratch_types=[
            pltpu.VMEM((LB,), jnp.int32),
            pltpu.VMEM((LB,), jnp.int32),
            pltpu.VMEM((LB, D), jnp.float32),
            pltpu.VMEM((LB, D), jnp.float32),
            pltpu.SemaphoreType.DMA,
            pltpu.SemaphoreType.DMA,
        ],
    )
    return kern(qf, kf, gidx_g, sidx_g)


def _alpha_body(qd_ref, ks_ref, s_ref):
    a = jnp.sum(qd_ref[...] * ks_ref[...], axis=1) * (1.0 / (D ** 0.5))
    s_ref[...] = jnp.exp(a)


def _alpha(qd, ks):
    return pl.pallas_call(
        _alpha_body,
        grid=(EP // _BC,),
        in_specs=[
            pl.BlockSpec((_BC, D), lambda i: (i, 0)),
            pl.BlockSpec((_BC, D), lambda i: (i, 0)),
        ],
        out_specs=pl.BlockSpec((_BC,), lambda i: (i,)),
        out_shape=jax.ShapeDtypeStruct((EP,), jnp.float32),
    )(qd, ks)


def _edge_scatter_body(vlo_hbm, vhi_hbm, gs_hbm, ss_hbm, s_hbm, out_hbm,
                       idx_v, si_v, s_v, rows, acc, sem):
    cid = lax.axis_index("c")
    sid = lax.axis_index("s")

    @pl.loop(0, LB)
    def _(e):
        for c in range(AW // 16):
            rows[e, pl.ds(16 * c, 16)] = jnp.zeros((16,), jnp.float32)

    @pl.loop(0, STRIPE // LB)
    def _(j):
        pltpu.sync_copy(rows, acc.at[pl.ds(sid * STRIPE + j * LB, LB)])

    plsc.subcore_barrier()

    base0 = sid * PER_SUB_D

    @pl.loop(0, PER_SUB_D // LB)
    def _(b):
        base = base0 + b * LB
        pltpu.sync_copy(gs_hbm.at[pl.ds(base, LB)], idx_v)
        pltpu.sync_copy(ss_hbm.at[pl.ds(base, LB)], si_v)
        pltpu.sync_copy(s_hbm.at[pl.ds(base, LB)], s_v)

        @pl.when(cid == 0)
        def _():
            pltpu.async_copy(vlo_hbm.at[si_v], rows, sem).wait()

        @pl.when(cid == 1)
        def _():
            pltpu.async_copy(vhi_hbm.at[si_v], rows, sem).wait()

        @pl.loop(0, LB)
        def _(e):
            e_idx = jnp.zeros((16,), jnp.int32) + e
            s_splat = plsc.load_gather(s_v, [e_idx])
            for c in range(AW // 16):
                sl = pl.ds(16 * c, 16)
                rows[e, sl] = rows[e, sl] * s_splat

        pltpu.sync_copy(rows, acc.at[idx_v], add=True)

    plsc.subcore_barrier()

    @pl.loop(0, STRIPE // LB)
    def _(j):
        r0 = sid * STRIPE + j * LB
        pltpu.sync_copy(acc.at[pl.ds(r0, LB)], out_hbm.at[cid, pl.ds(r0, LB)])


def _sc_compiler_params():
    cp = pltpu.CompilerParams()
    fields = pltpu.CompilerParams.__dataclass_fields__
    if "needs_layout_passes" in fields:
        cp = dataclasses.replace(cp, needs_layout_passes=False)
    if "use_tc_tiling_on_sc" in fields:
        cp = dataclasses.replace(cp, use_tc_tiling_on_sc=False)
    return cp


def _edge_scatter(vlo, vhi, gidx_s, sidx_g, s):
    mesh = plsc.VectorSubcoreMesh(core_axis_name="c", subcore_axis_name="s")
    kern = pl.kernel(
        _edge_scatter_body,
        mesh=mesh,
        out_type=jax.ShapeDtypeStruct((NC, ACC_ROWS, AW), jnp.float32),
        scratch_types=[
            pltpu.VMEM((LB,), jnp.int32),
            pltpu.VMEM((LB,), jnp.int32),
            pltpu.VMEM((LB,), jnp.float32),
            pltpu.VMEM((LB, AW), jnp.float32),
            pltpu.VMEM_SHARED((ACC_ROWS, AW), jnp.float32),
            pltpu.SemaphoreType.DMA,
        ],
        compiler_params=_sc_compiler_params(),
    )
    return kern(vlo, vhi, gidx_s, sidx_g, s)


def _combine_body(a0_ref, a1_ref, sk_ref, z_ref, cs_ref):
    i = pl.program_id(0)
    num = jnp.concatenate([a0_ref[:, :64], a1_ref[:, :64]], axis=1)
    den = a0_ref[:, 64:65]
    z = jnp.where(den > 0.0, num / jnp.where(den > 0.0, den, 1.0), 0.0)
    z = z + sk_ref[...]
    z_ref[...] = z
    r = i // (N // _BN)
    bsum = jnp.sum(z, axis=0, keepdims=True)
    rows2 = lax.broadcasted_iota(jnp.int32, (R, D), 0)
    contrib = jnp.where(rows2 == r, bsum, 0.0)

    @pl.when(i == 0)
    def _():
        cs_ref[...] = jnp.zeros((R, D), jnp.float32)

    cs_ref[...] += contrib


def _combine(acc0, acc1, skipf):
    return pl.pallas_call(
        _combine_body,
        grid=(2 * N // _BN,),
        in_specs=[
            pl.BlockSpec((_BN, AW), lambda i: (i, 0)),
            pl.BlockSpec((_BN, AW), lambda i: (i, 0)),
            pl.BlockSpec((_BN, D), lambda i: (i, 0)),
        ],
        out_specs=[
            pl.BlockSpec((_BN, D), lambda i: (i, 0)),
            pl.BlockSpec((R, D), lambda i: (0, 0)),
        ],
        out_shape=[
            jax.ShapeDtypeStruct((2 * N, D), jnp.float32),
            jax.ShapeDtypeStruct((R, D), jnp.float32),
        ],
    )(acc0, acc1, skipf)


def _sem_body(z_ref, cs_ref, wat_ref, a_ref, o_ref):
    t = cs_ref[...] * (1.0 / N)
    w = jnp.dot(t, wat_ref[...], preferred_element_type=jnp.float32)
    a = a_ref[0, 0]
    w = jnp.where(w >= 0.0, w, a * w)
    m = jnp.max(w, axis=0, keepdims=True)
    ew = jnp.exp(w - m)
    beta = ew / jnp.sum(ew, axis=0, keepdims=True)
    c0 = jnp.sum(beta[0:1, :]) * (1.0 / SH)
    c1 = jnp.sum(beta[1:2, :]) * (1.0 / SH)
    o_ref[...] = c0 * z_ref[:N, :] + c1 * z_ref[N:, :]


def _semantic(z, cs, W_att, a_sem):
    return pl.pallas_call(
        _sem_body,
        grid=(1,),
        in_specs=[
            pl.BlockSpec((2 * N, D), lambda i: (0, 0)),
            pl.BlockSpec((R, D), lambda i: (0, 0)),
            pl.BlockSpec((D, SH), lambda i: (0, 0)),
            pl.BlockSpec((1, 1), lambda i: (0, 0)),
        ],
        out_specs=pl.BlockSpec((N, D), lambda i: (0, 0)),
        out_shape=jax.ShapeDtypeStruct((N, D), jnp.float32),
    )(z, cs, W_att, a_sem)


def kernel(x, edge_index, edge_type, Wq, bq, Wk, bk, Wv, bv, Ws, bs, W_att, a_sem):
    src = edge_index[0].astype(jnp.int32)
    dst = edge_index[1].astype(jnp.int32)
    et = edge_type.astype(jnp.int32)

    gidx = et * N + dst
    sidx = et * N + src
    pad = EP - E
    gidx_g = jnp.concatenate([gidx, jnp.zeros((pad,), jnp.int32)])
    sidx_g = jnp.concatenate([sidx, jnp.zeros((pad,), jnp.int32)])
    gidx_s = jnp.concatenate([gidx, jnp.full((pad,), DUMMY, jnp.int32)])

    Wall = jnp.stack([Wq, Wk, Wv, Ws])
    ball = jnp.stack([bq, bk, bv, bs])[:, :, None, :]

    proj = _projections(x, Wall, ball)
    qf = proj[0].reshape(R * N, D)
    kf = proj[1].reshape(R * N, D)
    vf = proj[2].reshape(R * N, D)
    skipf = proj[3].reshape(R * N, D)

    qd, ks = _gather_qk(qf, kf, gidx_g, sidx_g)
    s = _alpha(qd, ks)

    ones = jnp.ones((R * N, 1), jnp.float32)
    zpad = jnp.zeros((R * N, AW - 65), jnp.float32)
    vlo = jnp.concatenate([vf[:, :64], ones, zpad], axis=1)
    vhi = jnp.concatenate([vf[:, 64:], ones, zpad], axis=1)

    acc = _edge_scatter(vlo, vhi, gidx_s, sidx_g, s)

    z, cs = _combine(acc[0], acc[1], skipf)
    return _semantic(z, cs, W_att, a_sem.reshape(1, 1).astype(jnp.float32))


# speedup vs baseline: 9.5718x; 1.2764x over previous
import dataclasses
import functools

import jax
import jax.numpy as jnp
from jax import lax
from jax.experimental import pallas as pl
from jax.experimental.pallas import tpu as pltpu
from jax.experimental.pallas import tpu_sc as plsc

N = 10000
E = 320000
D = 128
R = 2
SH = 4

NC = 2
NS = 16
NW = NC * NS
LB = 128
EP = 327680
PER_TILE_B = EP // NW
PER_SUB_D = EP // NS
ACC_ROWS = 20480
STRIPE = ACC_ROWS // NS
DUMMY = 2 * N
AW = 80

_BN = 1000
_BC = 2048


def _proj_body(x_ref, w_ref, b_ref, o_ref):
    o_ref[0, 0] = (
        jnp.dot(x_ref[...], w_ref[0, 0], preferred_element_type=jnp.float32)
        + b_ref[0, 0]
    )


def _projections(x, Wall, ball):
    return pl.pallas_call(
        _proj_body,
        grid=(4, R, N // _BN),
        in_specs=[
            pl.BlockSpec((_BN, D), lambda k, r, i: (i, 0)),
            pl.BlockSpec((1, 1, D, D), lambda k, r, i: (k, r, 0, 0)),
            pl.BlockSpec((1, 1, 1, D), lambda k, r, i: (k, r, 0, 0)),
        ],
        out_specs=pl.BlockSpec((1, 1, _BN, D), lambda k, r, i: (k, r, i, 0)),
        out_shape=jax.ShapeDtypeStruct((4, R, N, D), jnp.float32),
    )(x, Wall, ball)


_NB_B = PER_TILE_B // LB


def _gather_qk_body(qf_hbm, kf_hbm, cidx_hbm, qd_hbm, ks_hbm,
                    cidx_v, qr0, kr0, qr1, kr1, gsem0, gsem1, wsem0, wsem1):
    cid = lax.axis_index("c")
    sid = lax.axis_index("s")
    wid = sid * NC + cid
    base0 = wid * PER_TILE_B

    pltpu.sync_copy(cidx_hbm.at[pl.ds(wid * 2 * _NB_B, 2 * _NB_B)], cidx_v)

    def gather_start(b, qr, kr, gsem):
        pltpu.async_copy(qf_hbm.at[cidx_v.at[2 * b]], qr, gsem)
        pltpu.async_copy(kf_hbm.at[cidx_v.at[2 * b + 1]], kr, gsem)

    def gather_wait(b, qr, kr, gsem):
        pltpu.make_async_copy(qf_hbm.at[cidx_v.at[2 * b]], qr, gsem).wait()
        pltpu.make_async_copy(kf_hbm.at[cidx_v.at[2 * b + 1]], kr, gsem).wait()

    def write_start(b, qr, kr, wsem):
        pltpu.async_copy(qr, qd_hbm.at[pl.ds(base0 + b * LB, LB)], wsem)
        pltpu.async_copy(kr, ks_hbm.at[pl.ds(base0 + b * LB, LB)], wsem)

    def write_wait(b, qr, kr, wsem):
        pltpu.make_async_copy(qr, qd_hbm.at[pl.ds(base0 + b * LB, LB)], wsem).wait()
        pltpu.make_async_copy(kr, ks_hbm.at[pl.ds(base0 + b * LB, LB)], wsem).wait()

    gather_start(0, qr0, kr0, gsem0)

    @pl.loop(0, _NB_B // 2)
    def _(t):
        c0 = 2 * t

        @pl.when(t > 0)
        def _():
            write_wait(c0 - 1, qr1, kr1, wsem1)

        gather_start(c0 + 1, qr1, kr1, gsem1)
        gather_wait(c0, qr0, kr0, gsem0)
        write_start(c0, qr0, kr0, wsem0)

        @pl.when(t + 1 < _NB_B // 2)
        def _():
            write_wait(c0, qr0, kr0, wsem0)
            gather_start(c0 + 2, qr0, kr0, gsem0)

        gather_wait(c0 + 1, qr1, kr1, gsem1)
        write_start(c0 + 1, qr1, kr1, wsem1)

    write_wait(_NB_B - 2, qr0, kr0, wsem0)
    write_wait(_NB_B - 1, qr1, kr1, wsem1)


def _gather_qk(qf, kf, cidxB):
    mesh = plsc.VectorSubcoreMesh(core_axis_name="c", subcore_axis_name="s")
    kern = pl.kernel(
        _gather_qk_body,
        mesh=mesh,
        out_type=[
            jax.ShapeDtypeStruct((EP, D), jnp.float32),
            jax.ShapeDtypeStruct((EP, D), jnp.float32),
        ],
        scratch_types=[
            pltpu.VMEM((2 * _NB_B, LB), jnp.int32),
            pltpu.VMEM((LB, D), jnp.float32),
            pltpu.VMEM((LB, D), jnp.float32),
            pltpu.VMEM((LB, D), jnp.float32),
            pltpu.VMEM((LB, D), jnp.float32),
            pltpu.SemaphoreType.DMA,
            pltpu.SemaphoreType.DMA,
            pltpu.SemaphoreType.DMA,
            pltpu.SemaphoreType.DMA,
        ],
        compiler_params=_sc_compiler_params(),
    )
    return kern(qf, kf, cidxB)


def _alpha_body(qd_ref, ks_ref, s_ref):
    a = jnp.sum(qd_ref[...] * ks_ref[...], axis=1) * (1.0 / (D ** 0.5))
    s_ref[...] = jnp.exp(a)


def _alpha(qd, ks):
    return pl.pallas_call(
        _alpha_body,
        grid=(EP // _BC,),
        in_specs=[
            pl.BlockSpec((_BC, D), lambda i: (i, 0)),
            pl.BlockSpec((_BC, D), lambda i: (i, 0)),
        ],
        out_specs=pl.BlockSpec((_BC,), lambda i: (i,)),
        out_shape=jax.ShapeDtypeStruct((EP,), jnp.float32),
    )(qd, ks)


_NB_D = PER_SUB_D // LB


def _edge_scatter_body(vlo_hbm, vhi_hbm, cidx_hbm, s_hbm, out_hbm,
                       c0buf, c1buf, s0buf, s1buf, r0, r1, acc,
                       gsem0, gsem1, isem0, isem1):
    cid = lax.axis_index("c")
    sid = lax.axis_index("s")

    @pl.loop(0, LB, step=8)
    def _(e):
        for k in range(8):
            for c in range(AW // 16):
                r0[e + k, pl.ds(16 * c, 16)] = jnp.zeros((16,), jnp.float32)

    @pl.loop(0, STRIPE // LB)
    def _(j):
        pltpu.sync_copy(r0, acc.at[pl.ds(sid * STRIPE + j * LB, LB)])

    plsc.subcore_barrier()

    cbufs = (c0buf, c1buf)
    sbufs = (s0buf, s1buf)
    rbufs = (r0, r1)
    gsems = (gsem0, gsem1)
    isems = (isem0, isem1)

    def load_start(b, p):
        pltpu.async_copy(cidx_hbm.at[pl.ds(sid * 2 * _NB_D + 2 * b, 2)],
                         cbufs[p], isems[p])
        pltpu.async_copy(s_hbm.at[pl.ds(sid * PER_SUB_D + b * LB, LB)],
                         sbufs[p], isems[p])

    def load_wait(b, p):
        pltpu.make_async_copy(cidx_hbm.at[pl.ds(sid * 2 * _NB_D + 2 * b, 2)],
                              cbufs[p], isems[p]).wait()
        pltpu.make_async_copy(s_hbm.at[pl.ds(sid * PER_SUB_D + b * LB, LB)],
                              sbufs[p], isems[p]).wait()

    def gather_start(p):
        @pl.when(cid == 0)
        def _():
            pltpu.async_copy(vlo_hbm.at[cbufs[p].at[1]], rbufs[p], gsems[p])

        @pl.when(cid == 1)
        def _():
            pltpu.async_copy(vhi_hbm.at[cbufs[p].at[1]], rbufs[p], gsems[p])

    def gather_wait(p):
        @pl.when(cid == 0)
        def _():
            pltpu.make_async_copy(vlo_hbm.at[cbufs[p].at[1]], rbufs[p],
                                  gsems[p]).wait()

        @pl.when(cid == 1)
        def _():
            pltpu.make_async_copy(vhi_hbm.at[cbufs[p].at[1]], rbufs[p],
                                  gsems[p]).wait()

    def scale_scatter(p):
        rbuf = rbufs[p]
        sbuf = sbufs[p]

        @pl.loop(0, LB, step=4)
        def _(e):
            for k in range(4):
                e_idx = jnp.zeros((16,), jnp.int32) + (e + k)
                s_splat = plsc.load_gather(sbuf, [e_idx])
                for c in range(AW // 16):
                    sl = pl.ds(16 * c, 16)
                    rbuf[e + k, sl] = rbuf[e + k, sl] * s_splat

        pltpu.sync_copy(rbuf, acc.at[cbufs[p].at[0]], add=True)

    load_start(0, 0)
    load_wait(0, 0)
    gather_start(0)
    load_start(1, 1)

    @pl.loop(0, _NB_D // 2)
    def _(t):
        for p in (0, 1):
            c = 2 * t + p
            gather_wait(p)
            if p == 0:
                load_wait(c + 1, 1)
                gather_start(1)
            else:
                @pl.when(t + 1 < _NB_D // 2)
                def _():
                    load_wait(c + 1, 0)
                    gather_start(0)

            scale_scatter(p)

            if p == 0:

                @pl.when(t + 1 < _NB_D // 2)
                def _():
                    load_start(c + 2, 0)
            else:

                @pl.when(t + 1 < _NB_D // 2)
                def _():
                    load_start(c + 2, 1)

    plsc.subcore_barrier()

    @pl.loop(0, STRIPE // LB)
    def _(j):
        rr = sid * STRIPE + j * LB
        pltpu.sync_copy(acc.at[pl.ds(rr, LB)], out_hbm.at[cid, pl.ds(rr, LB)])


def _sc_compiler_params():
    cp = pltpu.CompilerParams()
    fields = pltpu.CompilerParams.__dataclass_fields__
    if "needs_layout_passes" in fields:
        cp = dataclasses.replace(cp, needs_layout_passes=False)
    if "use_tc_tiling_on_sc" in fields:
        cp = dataclasses.replace(cp, use_tc_tiling_on_sc=False)
    return cp


def _edge_scatter(vlo, vhi, cidxD, s):
    mesh = plsc.VectorSubcoreMesh(core_axis_name="c", subcore_axis_name="s")
    kern = pl.kernel(
        _edge_scatter_body,
        mesh=mesh,
        out_type=jax.ShapeDtypeStruct((NC, ACC_ROWS, AW), jnp.float32),
        scratch_types=[
            pltpu.VMEM((2, LB), jnp.int32),
            pltpu.VMEM((2, LB), jnp.int32),
            pltpu.VMEM((LB,), jnp.float32),
            pltpu.VMEM((LB,), jnp.float32),
            pltpu.VMEM((LB, AW), jnp.float32),
            pltpu.VMEM((LB, AW), jnp.float32),
            pltpu.VMEM_SHARED((ACC_ROWS, AW), jnp.float32),
            pltpu.SemaphoreType.DMA,
            pltpu.SemaphoreType.DMA,
            pltpu.SemaphoreType.DMA,
            pltpu.SemaphoreType.DMA,
        ],
        compiler_params=_sc_compiler_params(),
    )
    return kern(vlo, vhi, cidxD, s)


def _combine_body(a0_ref, a1_ref, sk_ref, z_ref, cs_ref):
    i = pl.program_id(0)
    num = jnp.concatenate([a0_ref[:, :64], a1_ref[:, :64]], axis=1)
    den = a0_ref[:, 64:65]
    z = jnp.where(den > 0.0, num / jnp.where(den > 0.0, den, 1.0), 0.0)
    z = z + sk_ref[...]
    z_ref[...] = z
    r = i // (N // _BN)
    bsum = jnp.sum(z, axis=0, keepdims=True)
    rows2 = lax.broadcasted_iota(jnp.int32, (R, D), 0)
    contrib = jnp.where(rows2 == r, bsum, 0.0)

    @pl.when(i == 0)
    def _():
        cs_ref[...] = jnp.zeros((R, D), jnp.float32)

    cs_ref[...] += contrib


def _combine(acc0, acc1, skipf):
    return pl.pallas_call(
        _combine_body,
        grid=(2 * N // _BN,),
        in_specs=[
            pl.BlockSpec((_BN, AW), lambda i: (i, 0)),
            pl.BlockSpec((_BN, AW), lambda i: (i, 0)),
            pl.BlockSpec((_BN, D), lambda i: (i, 0)),
        ],
        out_specs=[
            pl.BlockSpec((_BN, D), lambda i: (i, 0)),
            pl.BlockSpec((R, D), lambda i: (0, 0)),
        ],
        out_shape=[
            jax.ShapeDtypeStruct((2 * N, D), jnp.float32),
            jax.ShapeDtypeStruct((R, D), jnp.float32),
        ],
    )(acc0, acc1, skipf)


def _sem_body(z_ref, cs_ref, wat_ref, a_ref, o_ref):
    t = cs_ref[...] * (1.0 / N)
    w = jnp.dot(t, wat_ref[...], preferred_element_type=jnp.float32)
    a = a_ref[0, 0]
    w = jnp.where(w >= 0.0, w, a * w)
    m = jnp.max(w, axis=0, keepdims=True)
    ew = jnp.exp(w - m)
    beta = ew / jnp.sum(ew, axis=0, keepdims=True)
    c0 = jnp.sum(beta[0:1, :]) * (1.0 / SH)
    c1 = jnp.sum(beta[1:2, :]) * (1.0 / SH)
    o_ref[...] = c0 * z_ref[:N, :] + c1 * z_ref[N:, :]


def _semantic(z, cs, W_att, a_sem):
    return pl.pallas_call(
        _sem_body,
        grid=(1,),
        in_specs=[
            pl.BlockSpec((2 * N, D), lambda i: (0, 0)),
            pl.BlockSpec((R, D), lambda i: (0, 0)),
            pl.BlockSpec((D, SH), lambda i: (0, 0)),
            pl.BlockSpec((1, 1), lambda i: (0, 0)),
        ],
        out_specs=pl.BlockSpec((N, D), lambda i: (0, 0)),
        out_shape=jax.ShapeDtypeStruct((N, D), jnp.float32),
    )(z, cs, W_att, a_sem)


def kernel(x, edge_index, edge_type, Wq, bq, Wk, bk, Wv, bv, Ws, bs, W_att, a_sem):
    src = edge_index[0].astype(jnp.int32)
    dst = edge_index[1].astype(jnp.int32)
    et = edge_type.astype(jnp.int32)

    gidx = et * N + dst
    sidx = et * N + src
    pad = EP - E
    gidx_g = jnp.concatenate([gidx, jnp.zeros((pad,), jnp.int32)])
    sidx_g = jnp.concatenate([sidx, jnp.zeros((pad,), jnp.int32)])
    gidx_s = jnp.concatenate([gidx, jnp.full((pad,), DUMMY, jnp.int32)])

    cidxB = jnp.stack(
        [gidx_g.reshape(NW, _NB_B, LB), sidx_g.reshape(NW, _NB_B, LB)], axis=2
    ).reshape(NW * 2 * _NB_B, LB)
    cidxD = jnp.stack(
        [gidx_s.reshape(NS, _NB_D, LB), sidx_g.reshape(NS, _NB_D, LB)], axis=2
    ).reshape(NS * 2 * _NB_D, LB)

    Wall = jnp.stack([Wq, Wk, Wv, Ws])
    ball = jnp.stack([bq, bk, bv, bs])[:, :, None, :]

    proj = _projections(x, Wall, ball)
    qf = proj[0].reshape(R * N, D)
    kf = proj[1].reshape(R * N, D)
    vf = proj[2].reshape(R * N, D)
    skipf = proj[3].reshape(R * N, D)

    qd, ks = _gather_qk(qf, kf, cidxB)
    s = _alpha(qd, ks)

    ones = jnp.ones((R * N, 1), jnp.float32)
    zpad = jnp.zeros((R * N, AW - 65), jnp.float32)
    vlo = jnp.concatenate([vf[:, :64], ones, zpad], axis=1)
    vhi = jnp.concatenate([vf[:, 64:], ones, zpad], axis=1)

    acc = _edge_scatter(vlo, vhi, cidxD, s)

    z, cs = _combine(acc[0], acc[1], skipf)
    return _semantic(z, cs, W_att, a_sem.reshape(1, 1).astype(jnp.float32))


# speedup vs baseline: 11.8094x; 1.2338x over previous
import dataclasses
import functools

import jax
import jax.numpy as jnp
from jax import lax
from jax.experimental import pallas as pl
from jax.experimental.pallas import tpu as pltpu
from jax.experimental.pallas import tpu_sc as plsc

N = 10000
E = 320000
D = 128
R = 2
SH = 4

NC = 2
NS = 16
NW = NC * NS
LB = 128
EP = 327680
PER_TILE_B = EP // NW
PER_SUB_D = EP // NS
ACC_ROWS = 20480
STRIPE = ACC_ROWS // NS
DUMMY = 2 * N
AW = 80

_BN = 1000
_BC = 2048


def _proj_body(x_ref, w_ref, b_ref, o_ref):
    o_ref[0, 0] = (
        jnp.dot(x_ref[...], w_ref[0, 0], preferred_element_type=jnp.float32)
        + b_ref[0, 0]
    )


def _projections(x, Wall, ball):
    return pl.pallas_call(
        _proj_body,
        grid=(4, R, N // _BN),
        in_specs=[
            pl.BlockSpec((_BN, D), lambda k, r, i: (i, 0)),
            pl.BlockSpec((1, 1, D, D), lambda k, r, i: (k, r, 0, 0)),
            pl.BlockSpec((1, 1, 1, D), lambda k, r, i: (k, r, 0, 0)),
        ],
        out_specs=pl.BlockSpec((1, 1, _BN, D), lambda k, r, i: (k, r, i, 0)),
        out_shape=jax.ShapeDtypeStruct((4, R, N, D), jnp.float32),
    )(x, Wall, ball)


_NB_B = PER_TILE_B // LB
_INV_SQRT_D = 1.0 / (D ** 0.5)


def _gather_dot_body(qf_hbm, kf_hbm, cidx_hbm, s_hbm,
                     cidx_v, qr0, kr0, qr1, kr1, dot_v, sbat0, sbat1,
                     gsem0, gsem1, wsem0, wsem1):
    cid = lax.axis_index("c")
    sid = lax.axis_index("s")
    wid = sid * NC + cid
    base0 = wid * PER_TILE_B

    pltpu.sync_copy(cidx_hbm.at[pl.ds(wid * 2 * _NB_B, 2 * _NB_B)], cidx_v)

    def gather_start(b, qr, kr, gsem):
        pltpu.async_copy(qf_hbm.at[cidx_v.at[2 * b]], qr, gsem)
        pltpu.async_copy(kf_hbm.at[cidx_v.at[2 * b + 1]], kr, gsem)

    def gather_wait(b, qr, kr, gsem):
        pltpu.make_async_copy(qf_hbm.at[cidx_v.at[2 * b]], qr, gsem).wait()
        pltpu.make_async_copy(kf_hbm.at[cidx_v.at[2 * b + 1]], kr, gsem).wait()

    def compute_s(qr, kr, sbat):
        @pl.loop(0, LB, step=2)
        def _(e):
            for k in (0, 1):
                ek = e + k
                a = qr[ek, pl.ds(0, 16)] * kr[ek, pl.ds(0, 16)]
                for c in range(1, D // 16):
                    a = a + qr[ek, pl.ds(16 * c, 16)] * kr[ek, pl.ds(16 * c, 16)]
                dot_v[pl.ds(ek * 16, 16)] = a

        for g in range(LB // 16):
            cols0 = lax.iota(jnp.int32, 16) * 16 + g * 256
            tot = plsc.load_gather(dot_v, [cols0])
            for c in range(1, 16):
                tot = tot + plsc.load_gather(dot_v, [cols0 + c])
            sbat[pl.ds(g * 16, 16)] = jnp.exp(tot * _INV_SQRT_D)

    def swrite_start(b, sbat, wsem):
        pltpu.async_copy(sbat, s_hbm.at[pl.ds(base0 + b * LB, LB)], wsem)

    def swrite_wait(b, sbat, wsem):
        pltpu.make_async_copy(sbat, s_hbm.at[pl.ds(base0 + b * LB, LB)],
                              wsem).wait()

    gather_start(0, qr0, kr0, gsem0)

    @pl.loop(0, _NB_B // 2)
    def _(t):
        c0 = 2 * t
        gather_start(c0 + 1, qr1, kr1, gsem1)
        gather_wait(c0, qr0, kr0, gsem0)

        @pl.when(t > 0)
        def _():
            swrite_wait(c0 - 2, sbat0, wsem0)

        compute_s(qr0, kr0, sbat0)
        swrite_start(c0, sbat0, wsem0)

        @pl.when(t + 1 < _NB_B // 2)
        def _():
            gather_start(c0 + 2, qr0, kr0, gsem0)

        gather_wait(c0 + 1, qr1, kr1, gsem1)

        @pl.when(t > 0)
        def _():
            swrite_wait(c0 - 1, sbat1, wsem1)

        compute_s(qr1, kr1, sbat1)
        swrite_start(c0 + 1, sbat1, wsem1)

    swrite_wait(_NB_B - 2, sbat0, wsem0)
    swrite_wait(_NB_B - 1, sbat1, wsem1)


def _gather_dot(qf, kf, cidxB):
    mesh = plsc.VectorSubcoreMesh(core_axis_name="c", subcore_axis_name="s")
    kern = pl.kernel(
        _gather_dot_body,
        mesh=mesh,
        out_type=jax.ShapeDtypeStruct((EP,), jnp.float32),
        scratch_types=[
            pltpu.VMEM((2 * _NB_B, LB), jnp.int32),
            pltpu.VMEM((LB, D), jnp.float32),
            pltpu.VMEM((LB, D), jnp.float32),
            pltpu.VMEM((LB, D), jnp.float32),
            pltpu.VMEM((LB, D), jnp.float32),
            pltpu.VMEM((LB * 16,), jnp.float32),
            pltpu.VMEM((LB,), jnp.float32),
            pltpu.VMEM((LB,), jnp.float32),
            pltpu.SemaphoreType.DMA,
            pltpu.SemaphoreType.DMA,
            pltpu.SemaphoreType.DMA,
            pltpu.SemaphoreType.DMA,
        ],
        compiler_params=_sc_compiler_params(),
    )
    return kern(qf, kf, cidxB)


_NB_D = PER_SUB_D // LB


def _edge_scatter_body(vlo_hbm, vhi_hbm, cidx_hbm, s_hbm, out_hbm,
                       c0buf, c1buf, s0buf, s1buf, r0, r1, acc,
                       gsem0, gsem1, isem0, isem1):
    cid = lax.axis_index("c")
    sid = lax.axis_index("s")

    @pl.loop(0, LB, step=8)
    def _(e):
        for k in range(8):
            for c in range(AW // 16):
                r0[e + k, pl.ds(16 * c, 16)] = jnp.zeros((16,), jnp.float32)

    @pl.loop(0, STRIPE // LB)
    def _(j):
        pltpu.sync_copy(r0, acc.at[pl.ds(sid * STRIPE + j * LB, LB)])

    plsc.subcore_barrier()

    cbufs = (c0buf, c1buf)
    sbufs = (s0buf, s1buf)
    rbufs = (r0, r1)
    gsems = (gsem0, gsem1)
    isems = (isem0, isem1)

    def load_start(b, p):
        pltpu.async_copy(cidx_hbm.at[pl.ds(sid * 2 * _NB_D + 2 * b, 2)],
                         cbufs[p], isems[p])
        pltpu.async_copy(s_hbm.at[pl.ds(sid * PER_SUB_D + b * LB, LB)],
                         sbufs[p], isems[p])

    def load_wait(b, p):
        pltpu.make_async_copy(cidx_hbm.at[pl.ds(sid * 2 * _NB_D + 2 * b, 2)],
                              cbufs[p], isems[p]).wait()
        pltpu.make_async_copy(s_hbm.at[pl.ds(sid * PER_SUB_D + b * LB, LB)],
                              sbufs[p], isems[p]).wait()

    def gather_start(p):
        @pl.when(cid == 0)
        def _():
            pltpu.async_copy(vlo_hbm.at[cbufs[p].at[1]], rbufs[p], gsems[p])

        @pl.when(cid == 1)
        def _():
            pltpu.async_copy(vhi_hbm.at[cbufs[p].at[1]], rbufs[p], gsems[p])

    def gather_wait(p):
        @pl.when(cid == 0)
        def _():
            pltpu.make_async_copy(vlo_hbm.at[cbufs[p].at[1]], rbufs[p],
                                  gsems[p]).wait()

        @pl.when(cid == 1)
        def _():
            pltpu.make_async_copy(vhi_hbm.at[cbufs[p].at[1]], rbufs[p],
                                  gsems[p]).wait()

    def scale_scatter(p):
        rbuf = rbufs[p]
        sbuf = sbufs[p]

        @pl.loop(0, LB, step=4)
        def _(e):
            for k in range(4):
                e_idx = jnp.zeros((16,), jnp.int32) + (e + k)
                s_splat = plsc.load_gather(sbuf, [e_idx])
                for c in range(AW // 16):
                    sl = pl.ds(16 * c, 16)
                    rbuf[e + k, sl] = rbuf[e + k, sl] * s_splat

        pltpu.sync_copy(rbuf, acc.at[cbufs[p].at[0]], add=True)

    load_start(0, 0)
    load_wait(0, 0)
    gather_start(0)
    load_start(1, 1)

    @pl.loop(0, _NB_D // 2)
    def _(t):
        for p in (0, 1):
            c = 2 * t + p
            gather_wait(p)
            if p == 0:
                load_wait(c + 1, 1)
                gather_start(1)
            else:
                @pl.when(t + 1 < _NB_D // 2)
                def _():
                    load_wait(c + 1, 0)
                    gather_start(0)

            scale_scatter(p)

            if p == 0:

                @pl.when(t + 1 < _NB_D // 2)
                def _():
                    load_start(c + 2, 0)
            else:

                @pl.when(t + 1 < _NB_D // 2)
                def _():
                    load_start(c + 2, 1)

    plsc.subcore_barrier()

    @pl.loop(0, STRIPE // LB)
    def _(j):
        rr = sid * STRIPE + j * LB
        pltpu.sync_copy(acc.at[pl.ds(rr, LB)], out_hbm.at[cid, pl.ds(rr, LB)])


def _sc_compiler_params():
    cp = pltpu.CompilerParams()
    fields = pltpu.CompilerParams.__dataclass_fields__
    if "needs_layout_passes" in fields:
        cp = dataclasses.replace(cp, needs_layout_passes=False)
    if "use_tc_tiling_on_sc" in fields:
        cp = dataclasses.replace(cp, use_tc_tiling_on_sc=False)
    return cp


def _edge_scatter(vlo, vhi, cidxD, s):
    mesh = plsc.VectorSubcoreMesh(core_axis_name="c", subcore_axis_name="s")
    kern = pl.kernel(
        _edge_scatter_body,
        mesh=mesh,
        out_type=jax.ShapeDtypeStruct((NC, ACC_ROWS, AW), jnp.float32),
        scratch_types=[
            pltpu.VMEM((2, LB), jnp.int32),
            pltpu.VMEM((2, LB), jnp.int32),
            pltpu.VMEM((LB,), jnp.float32),
            pltpu.VMEM((LB,), jnp.float32),
            pltpu.VMEM((LB, AW), jnp.float32),
            pltpu.VMEM((LB, AW), jnp.float32),
            pltpu.VMEM_SHARED((ACC_ROWS, AW), jnp.float32),
            pltpu.SemaphoreType.DMA,
            pltpu.SemaphoreType.DMA,
            pltpu.SemaphoreType.DMA,
            pltpu.SemaphoreType.DMA,
        ],
        compiler_params=_sc_compiler_params(),
    )
    return kern(vlo, vhi, cidxD, s)


def _combine_body(a0_ref, a1_ref, sk_ref, z_ref, cs_ref):
    i = pl.program_id(0)
    num = jnp.concatenate([a0_ref[:, :64], a1_ref[:, :64]], axis=1)
    den = a0_ref[:, 64:65]
    z = jnp.where(den > 0.0, num / jnp.where(den > 0.0, den, 1.0), 0.0)
    z = z + sk_ref[...]
    z_ref[...] = z
    r = i // (N // _BN)
    bsum = jnp.sum(z, axis=0, keepdims=True)
    rows2 = lax.broadcasted_iota(jnp.int32, (R, D), 0)
    contrib = jnp.where(rows2 == r, bsum, 0.0)

    @pl.when(i == 0)
    def _():
        cs_ref[...] = jnp.zeros((R, D), jnp.float32)

    cs_ref[...] += contrib


def _combine(acc0, acc1, skipf):
    return pl.pallas_call(
        _combine_body,
        grid=(2 * N // _BN,),
        in_specs=[
            pl.BlockSpec((_BN, AW), lambda i: (i, 0)),
            pl.BlockSpec((_BN, AW), lambda i: (i, 0)),
            pl.BlockSpec((_BN, D), lambda i: (i, 0)),
        ],
        out_specs=[
            pl.BlockSpec((_BN, D), lambda i: (i, 0)),
            pl.BlockSpec((R, D), lambda i: (0, 0)),
        ],
        out_shape=[
            jax.ShapeDtypeStruct((2 * N, D), jnp.float32),
            jax.ShapeDtypeStruct((R, D), jnp.float32),
        ],
    )(acc0, acc1, skipf)


def _sem_body(z_ref, cs_ref, wat_ref, a_ref, o_ref):
    t = cs_ref[...] * (1.0 / N)
    w = jnp.dot(t, wat_ref[...], preferred_element_type=jnp.float32)
    a = a_ref[0, 0]
    w = jnp.where(w >= 0.0, w, a * w)
    m = jnp.max(w, axis=0, keepdims=True)
    ew = jnp.exp(w - m)
    beta = ew / jnp.sum(ew, axis=0, keepdims=True)
    c0 = jnp.sum(beta[0:1, :]) * (1.0 / SH)
    c1 = jnp.sum(beta[1:2, :]) * (1.0 / SH)
    o_ref[...] = c0 * z_ref[:N, :] + c1 * z_ref[N:, :]


def _semantic(z, cs, W_att, a_sem):
    return pl.pallas_call(
        _sem_body,
        grid=(1,),
        in_specs=[
            pl.BlockSpec((2 * N, D), lambda i: (0, 0)),
            pl.BlockSpec((R, D), lambda i: (0, 0)),
            pl.BlockSpec((D, SH), lambda i: (0, 0)),
            pl.BlockSpec((1, 1), lambda i: (0, 0)),
        ],
        out_specs=pl.BlockSpec((N, D), lambda i: (0, 0)),
        out_shape=jax.ShapeDtypeStruct((N, D), jnp.float32),
    )(z, cs, W_att, a_sem)


def kernel(x, edge_index, edge_type, Wq, bq, Wk, bk, Wv, bv, Ws, bs, W_att, a_sem):
    src = edge_index[0].astype(jnp.int32)
    dst = edge_index[1].astype(jnp.int32)
    et = edge_type.astype(jnp.int32)

    gidx = et * N + dst
    sidx = et * N + src
    pad = EP - E
    gidx_g = jnp.concatenate([gidx, jnp.zeros((pad,), jnp.int32)])
    sidx_g = jnp.concatenate([sidx, jnp.zeros((pad,), jnp.int32)])
    gidx_s = jnp.concatenate([gidx, jnp.full((pad,), DUMMY, jnp.int32)])

    cidxB = jnp.stack(
        [gidx_g.reshape(NW, _NB_B, LB), sidx_g.reshape(NW, _NB_B, LB)], axis=2
    ).reshape(NW * 2 * _NB_B, LB)
    cidxD = jnp.stack(
        [gidx_s.reshape(NS, _NB_D, LB), sidx_g.reshape(NS, _NB_D, LB)], axis=2
    ).reshape(NS * 2 * _NB_D, LB)

    Wall = jnp.stack([Wq, Wk, Wv, Ws])
    ball = jnp.stack([bq, bk, bv, bs])[:, :, None, :]

    proj = _projections(x, Wall, ball)
    qf = proj[0].reshape(R * N, D)
    kf = proj[1].reshape(R * N, D)
    vf = proj[2].reshape(R * N, D)
    skipf = proj[3].reshape(R * N, D)

    s = _gather_dot(qf, kf, cidxB)

    ones = jnp.ones((R * N, 1), jnp.float32)
    zpad = jnp.zeros((R * N, AW - 65), jnp.float32)
    vlo = jnp.concatenate([vf[:, :64], ones, zpad], axis=1)
    vhi = jnp.concatenate([vf[:, 64:], ones, zpad], axis=1)

    acc = _edge_scatter(vlo, vhi, cidxD, s)

    z, cs = _combine(acc[0], acc[1], skipf)
    return _semantic(z, cs, W_att, a_sem.reshape(1, 1).astype(jnp.float32))
